# baseline (speedup 1.0000x reference)
import jax
import jax.numpy as jnp
from jax import lax
from jax.experimental import pallas as pl
from jax.experimental.pallas import tpu as pltpu


_VMEM_LIMIT = 64 * 1024 * 1024

_TILE_Z = 512
_DEC = 1024


def _sched(t):
    m = ((t >= 3).astype(jnp.int32) + (t >= 8).astype(jnp.int32)
         + (t >= 15).astype(jnp.int32))
    r = t - (m * m + 2 * m)
    return m, r


def _dec_tile(m, r):
    r2 = jnp.maximum(r - 2, 0)
    on_row = r2 <= m
    di = jnp.where(on_row, m, r2 - m - 1)
    dj = jnp.where(on_row, r2, m)
    return di, dj


def _fused(adj, x, w_bf16):
    n = adj.shape[0]
    f = x.shape[1]
    h = w_bf16.shape[1]
    n_z = n // _TILE_Z
    n_dec = (n // _DEC) * (n // _DEC)
    num_steps = n_z + n_dec

    def body(x_ref, w_ref, adj_ref, o_ref, s_ref, z_ref):
        t = pl.program_id(0)
        m, r = _sched(t)

        @pl.when(t == 0)
        def _():
            s_ref[...] = jnp.dot(
                x_ref[...].astype(jnp.bfloat16), w_ref[...],
                preferred_element_type=jnp.float32,
            ).astype(jnp.bfloat16)

        @pl.when(r < 2)
        def _():
            k = 2 * m + r
            z_ref[pl.ds(k * _TILE_Z, _TILE_Z), :] = jnp.maximum(
                jnp.dot(
                    adj_ref[...].astype(jnp.bfloat16), s_ref[...],
                    preferred_element_type=jnp.float32,
                ),
                0.0,
            ).astype(jnp.bfloat16)

        @pl.when(r >= 2)
        def _():
            di, dj = _dec_tile(m, r)
            zr = z_ref[pl.ds(di * _DEC, _DEC), :]
            zc = z_ref[pl.ds(dj * _DEC, _DEC), :]
            logits = lax.dot_general(
                zr, zc,
                dimension_numbers=(((1,), (1,)), ((), ())),
                preferred_element_type=jnp.float32,
            )
            o_ref[...] = jax.nn.sigmoid(logits)

    def adj_map(t):
        m, r = _sched(t)
        return (2 * m + jnp.minimum(r, 1), 0)

    def out_map(t):
        m, r = _sched(t)
        return _dec_tile(m, r)

    return pl.pallas_call(
        body,
        out_shape=jax.ShapeDtypeStruct((n, n), jnp.float32),
        grid=(num_steps,),
        in_specs=[
            pl.BlockSpec((n, f), lambda t: (0, 0)),
            pl.BlockSpec((f, h), lambda t: (0, 0)),
            pl.BlockSpec((_TILE_Z, n), adj_map),
        ],
        out_specs=pl.BlockSpec((_DEC, _DEC), out_map),
        scratch_shapes=[
            pltpu.VMEM((n, h), jnp.bfloat16),
            pltpu.VMEM((n, h), jnp.bfloat16),
        ],
        compiler_params=pltpu.CompilerParams(
            dimension_semantics=("arbitrary",),
            vmem_limit_bytes=_VMEM_LIMIT,
        ),
    )(x, w_bf16, adj)


def kernel(atac_feature, scrna_feature, adj_norm, edge_attr, gc1_weight):
    del atac_feature, edge_attr

    x = scrna_feature.astype(jnp.float32)
    adj = adj_norm.astype(jnp.float32)
    w_bf16 = gc1_weight.astype(jnp.bfloat16)

    return _fused(adj, x, w_bf16)


# speedup vs baseline: 1.1112x; 1.1112x over previous
import jax
import jax.numpy as jnp
from jax import lax
from jax.experimental import pallas as pl
from jax.experimental.pallas import tpu as pltpu


_VMEM_LIMIT = 64 * 1024 * 1024

_TILE_Z = 512
_DEC_I = 2048
_DEC_J = 1024

_ADJ_SLAB = (0, 1, 2, 3, 3, 4, 4, 5, 5, 6, 7, 7, 7, 7, 7, 7)
_DEC_DI = (0, 0, 0, 0, 0, 0, 0, 0, 0, 0, 0, 0, 1, 1, 1, 1)
_DEC_DJ = (0, 0, 0, 0, 0, 1, 1, 2, 2, 3, 3, 3, 0, 1, 2, 3)
_IS_DEC = (0, 0, 0, 0, 1, 0, 1, 0, 1, 0, 0, 1, 1, 1, 1, 1)


def _lut(t, vals):
    out = jnp.int32(vals[-1])
    for idx in range(len(vals) - 2, -1, -1):
        out = jnp.where(t == idx, jnp.int32(vals[idx]), out)
    return out


def _fused(adj, x, w_bf16):
    n = adj.shape[0]
    f = x.shape[1]
    h = w_bf16.shape[1]

    def body(x_ref, w_ref, adj_ref, o_ref, s_ref, z_ref):
        t = pl.program_id(0)

        @pl.when(t == 0)
        def _():
            s_ref[...] = jnp.dot(
                x_ref[...].astype(jnp.bfloat16), w_ref[...],
                preferred_element_type=jnp.float32,
            ).astype(jnp.bfloat16)

        is_dec = _lut(t, _IS_DEC)

        @pl.when(is_dec == 0)
        def _():
            k = _lut(t, _ADJ_SLAB)
            z_ref[pl.ds(k * _TILE_Z, _TILE_Z), :] = jnp.maximum(
                jnp.dot(
                    adj_ref[...].astype(jnp.bfloat16), s_ref[...],
                    preferred_element_type=jnp.float32,
                ),
                0.0,
            ).astype(jnp.bfloat16)

        @pl.when(is_dec == 1)
        def _():
            di = _lut(t, _DEC_DI)
            dj = _lut(t, _DEC_DJ)
            zr = z_ref[pl.ds(di * _DEC_I, _DEC_I), :]
            zc = z_ref[pl.ds(dj * _DEC_J, _DEC_J), :]
            logits = lax.dot_general(
                zr, zc,
                dimension_numbers=(((1,), (1,)), ((), ())),
                preferred_element_type=jnp.float32,
            )
            o_ref[...] = jax.nn.sigmoid(logits)

    def adj_map(t):
        return (_lut(t, _ADJ_SLAB), 0)

    def out_map(t):
        return (_lut(t, _DEC_DI), _lut(t, _DEC_DJ))

    return pl.pallas_call(
        body,
        out_shape=jax.ShapeDtypeStruct((n, n), jnp.float32),
        grid=(len(_ADJ_SLAB),),
        in_specs=[
            pl.BlockSpec((n, f), lambda t: (0, 0)),
            pl.BlockSpec((f, h), lambda t: (0, 0)),
            pl.BlockSpec((_TILE_Z, n), adj_map),
        ],
        out_specs=pl.BlockSpec((_DEC_I, _DEC_J), out_map),
        scratch_shapes=[
            pltpu.VMEM((n, h), jnp.bfloat16),
            pltpu.VMEM((n, h), jnp.bfloat16),
        ],
        compiler_params=pltpu.CompilerParams(
            dimension_semantics=("arbitrary",),
            vmem_limit_bytes=_VMEM_LIMIT,
        ),
    )(x, w_bf16, adj)


def kernel(atac_feature, scrna_feature, adj_norm, edge_attr, gc1_weight):
    del atac_feature, edge_attr

    x = scrna_feature.astype(jnp.float32)
    adj = adj_norm.astype(jnp.float32)
    w_bf16 = gc1_weight.astype(jnp.bfloat16)

    return _fused(adj, x, w_bf16)


# speedup vs baseline: 1.2446x; 1.1201x over previous
import jax
import jax.numpy as jnp
from jax import lax
from jax.experimental import pallas as pl
from jax.experimental.pallas import tpu as pltpu


_VMEM_LIMIT = 64 * 1024 * 1024

_TILE_Z = 512
_DEC_I = 2048
_DEC_J = 1024


def _fused(adj, x, w_bf16):
    n = adj.shape[0]
    f = x.shape[1]
    h = w_bf16.shape[1]
    n_z = n // _TILE_Z
    n_i = n // _DEC_I
    n_j = n // _DEC_J
    n_dec = n_i * n_j

    def body(x_ref, w_ref, adj_ref, o_ref, s_ref, z_ref):
        t = pl.program_id(0)

        @pl.when(t == 0)
        def _():
            s_ref[...] = jnp.dot(
                x_ref[...].astype(jnp.bfloat16), w_ref[...],
                preferred_element_type=jnp.float32,
            ).astype(jnp.bfloat16)

        @pl.when(t < n_z)
        def _():
            z_ref[pl.ds(t * _TILE_Z, _TILE_Z), :] = jnp.maximum(
                jnp.dot(
                    adj_ref[...].astype(jnp.bfloat16), s_ref[...],
                    preferred_element_type=jnp.float32,
                ),
                0.0,
            ).astype(jnp.bfloat16)

        @pl.when(t >= n_z)
        def _():
            d = t - n_z
            di = d // n_j
            dj = d % n_j
            zr = z_ref[pl.ds(di * _DEC_I, _DEC_I), :]
            zc = z_ref[pl.ds(dj * _DEC_J, _DEC_J), :]
            logits = lax.dot_general(
                zr, zc,
                dimension_numbers=(((1,), (1,)), ((), ())),
                preferred_element_type=jnp.float32,
            )
            o_ref[...] = 0.5 + 0.5 * jnp.tanh(0.5 * logits)

    def adj_map(t):
        return (jnp.minimum(t, n_z - 1), 0)

    def out_map(t):
        d = jnp.maximum(t - n_z, 0)
        return (d // n_j, d % n_j)

    return pl.pallas_call(
        body,
        out_shape=jax.ShapeDtypeStruct((n, n), jnp.float32),
        grid=(n_z + n_dec,),
        in_specs=[
            pl.BlockSpec((n, f), lambda t: (0, 0)),
            pl.BlockSpec((f, h), lambda t: (0, 0)),
            pl.BlockSpec((_TILE_Z, n), adj_map),
        ],
        out_specs=pl.BlockSpec((_DEC_I, _DEC_J), out_map),
        scratch_shapes=[
            pltpu.VMEM((n, h), jnp.bfloat16),
            pltpu.VMEM((n, h), jnp.bfloat16),
        ],
        compiler_params=pltpu.CompilerParams(
            dimension_semantics=("arbitrary",),
            vmem_limit_bytes=_VMEM_LIMIT,
        ),
    )(x, w_bf16, adj)


def kernel(atac_feature, scrna_feature, adj_norm, edge_attr, gc1_weight):
    del atac_feature, edge_attr

    x = scrna_feature.astype(jnp.float32)
    adj = adj_norm.astype(jnp.float32)
    w_bf16 = gc1_weight.astype(jnp.bfloat16)

    return _fused(adj, x, w_bf16)
